# Initial kernel scaffold; baseline (speedup 1.0000x reference)
import jax
import jax.numpy as jnp
from jax.experimental import pallas as pl


def kernel(x, edge_index, edge_weight, W1, b1, W2, b2, W3, b3):
    raise NotImplementedError("write your pallas kernel here")


# speedup vs baseline: 6.5312x; 6.5312x over previous
import functools

import jax
import jax.numpy as jnp
from jax import lax
from jax.experimental import pallas as pl
from jax.experimental.pallas import tpu as pltpu
from jax.experimental.pallas import tpu_sc as plsc

N = 10000
NP = 10240
E = 320000
D = 128
C = 128
RPT = 80
EP = 32 * RPT * C
RB = 2048


def _deg_body(dst_hbm, ew_hbm, z1_hbm, degp0_hbm, degp1_hbm, degsh, didx, ewv):
    c = lax.axis_index("c")
    s = lax.axis_index("s")
    wid = c * 16 + s
    pltpu.sync_copy(z1_hbm.at[pl.ds(s * 640, 640)], degsh.at[pl.ds(s * 640, 640)])
    plsc.subcore_barrier()
    base = wid * RPT
    pltpu.sync_copy(dst_hbm.at[pl.ds(base, RPT)], didx)
    pltpu.sync_copy(ew_hbm.at[pl.ds(base, RPT)], ewv)

    def step(k, carry):
        pltpu.sync_copy(ewv.at[k], degsh.at[didx.at[k]], add=True)
        return carry

    lax.fori_loop(0, RPT, step, 0)
    plsc.subcore_barrier()

    @pl.when(c == 0)
    def _():
        pltpu.sync_copy(degsh.at[pl.ds(s * 640, 640)],
                        degp0_hbm.at[pl.ds(s * 640, 640)])

    @pl.when(c == 1)
    def _():
        pltpu.sync_copy(degsh.at[pl.ds(s * 640, 640)],
                        degp1_hbm.at[pl.ds(s * 640, 640)])


_DNUMS = lax.GatherDimensionNumbers(
    offset_dims=(), collapsed_slice_dims=(0,), start_index_map=(0,))


def _bcast(v, l):
    return lax.gather(v, jnp.full((16, 1), l, jnp.int32), _DNUMS, (1,),
                      mode=lax.GatherScatterMode.PROMISE_IN_BOUNDS)


def _msg_body(y_hbm, src_hbm, dst_hbm, ew_hbm, z2_hbm, parts_hbm,
              accsh, sidx, didx, ewv, rows, gsem):
    c = lax.axis_index("c")
    s = lax.axis_index("s")
    wid = c * 16 + s
    pltpu.sync_copy(z2_hbm, accsh.at[pl.ds(s * 640, 640)])
    plsc.subcore_barrier()
    base = wid * RPT
    pltpu.sync_copy(src_hbm.at[pl.ds(base, RPT)], sidx)
    pltpu.sync_copy(dst_hbm.at[pl.ds(base, RPT)], didx)
    pltpu.sync_copy(ew_hbm.at[pl.ds(base, RPT)], ewv)

    def chunk(k, carry):
        pltpu.async_copy(y_hbm.at[sidx.at[k]], rows, gsem).wait()

        def scale(g, c2):
            ewg = ewv[k, pl.ds(g * 16, 16)]
            for l in range(16):
                ewj = _bcast(ewg, l)
                j = g * 16 + l
                for kk in range(8):
                    rows[j, pl.ds(kk * 16, 16)] = rows[j, pl.ds(kk * 16, 16)] * ewj
            return c2

        lax.fori_loop(0, C // 16, scale, 0)
        pltpu.sync_copy(rows, accsh.at[didx.at[k]], add=True)
        return carry

    lax.fori_loop(0, RPT, chunk, 0)
    plsc.subcore_barrier()
    pltpu.sync_copy(accsh.at[pl.ds(s * 640, 640)],
                    parts_hbm.at[c, pl.ds(s * 640, 640)])


@functools.cache
def _deg_kernel():
    mesh = plsc.VectorSubcoreMesh(core_axis_name="c", subcore_axis_name="s")
    return pl.kernel(
        _deg_body,
        out_type=(jax.ShapeDtypeStruct((NP,), jnp.float32),
                  jax.ShapeDtypeStruct((NP,), jnp.float32)),
        mesh=mesh,
        scratch_types=[
            pltpu.VMEM_SHARED((NP,), jnp.float32),
            pltpu.VMEM((RPT, C), jnp.int32),
            pltpu.VMEM((RPT, C), jnp.float32),
        ],
    )


@functools.cache
def _msg_kernel():
    mesh = plsc.VectorSubcoreMesh(core_axis_name="c", subcore_axis_name="s")
    return pl.kernel(
        _msg_body,
        out_type=jax.ShapeDtypeStruct((2, NP, D), jnp.float32),
        mesh=mesh,
        scratch_types=[
            pltpu.VMEM_SHARED((NP, D), jnp.float32),
            pltpu.VMEM((RPT, C), jnp.int32),
            pltpu.VMEM((RPT, C), jnp.int32),
            pltpu.VMEM((RPT, C), jnp.float32),
            pltpu.VMEM((C, D), jnp.float32),
            pltpu.SemaphoreType.DMA,
        ],
    )


def _k1_body(x_ref, w_ref, p0_ref, p1_ref, y_ref):
    dis = lax.rsqrt(p0_ref[...] + p1_ref[...] + 1.0)
    y_ref[...] = jnp.dot(x_ref[...], w_ref[...],
                         preferred_element_type=jnp.float32) * dis


def _kmid_body(a0_ref, a1_ref, yp_ref, p0_ref, p1_ref, b_ref, w_ref, y_ref):
    dis = lax.rsqrt(p0_ref[...] + p1_ref[...] + 1.0)
    h = jnp.maximum(
        dis * (a0_ref[...] + a1_ref[...] + yp_ref[...]) + b_ref[...], 0.0)
    y_ref[...] = jnp.dot(h, w_ref[...],
                         preferred_element_type=jnp.float32) * dis


def _kfin_body(a0_ref, a1_ref, yp_ref, p0_ref, p1_ref, b_ref, o_ref):
    dis = lax.rsqrt(p0_ref[...] + p1_ref[...] + 1.0)
    o_ref[...] = dis * (a0_ref[...] + a1_ref[...] + yp_ref[...]) + b_ref[...]


_row_spec = pl.BlockSpec((RB, D), lambda i: (i, 0))
_col_spec = pl.BlockSpec((RB, 1), lambda i: (i, 0))
_w_spec = pl.BlockSpec((D, D), lambda i: (0, 0))
_b_spec = pl.BlockSpec((1, D), lambda i: (0, 0))
_GRID = (NP // RB,)
_out_nd = jax.ShapeDtypeStruct((NP, D), jnp.float32)


@functools.cache
def _k1():
    return pl.pallas_call(
        _k1_body, grid=_GRID,
        in_specs=[_row_spec, _w_spec, _col_spec, _col_spec],
        out_specs=_row_spec, out_shape=_out_nd)


@functools.cache
def _kmid():
    return pl.pallas_call(
        _kmid_body, grid=_GRID,
        in_specs=[_row_spec, _row_spec, _row_spec, _col_spec, _col_spec,
                  _b_spec, _w_spec],
        out_specs=_row_spec, out_shape=_out_nd)


@functools.cache
def _kfin():
    return pl.pallas_call(
        _kfin_body, grid=_GRID,
        in_specs=[_row_spec, _row_spec, _row_spec, _col_spec, _col_spec,
                  _b_spec],
        out_specs=_row_spec, out_shape=_out_nd)


def kernel(x, edge_index, edge_weight, W1, b1, W2, b2, W3, b3):
    src, dst = edge_index[0], edge_index[1]
    pad_e = EP - E
    srcp = jnp.concatenate([src, jnp.zeros((pad_e,), src.dtype)]).reshape(-1, C)
    dstp = jnp.concatenate([dst, jnp.zeros((pad_e,), dst.dtype)]).reshape(-1, C)
    ewp = jnp.concatenate(
        [edge_weight, jnp.zeros((pad_e,), edge_weight.dtype)]).reshape(-1, C)
    xp = jnp.pad(x, ((0, NP - N), (0, 0)))
    z1 = jnp.zeros((NP,), jnp.float32)
    z2 = jnp.zeros((640, D), jnp.float32)

    degp0, degp1 = _deg_kernel()(dstp, ewp, z1)
    p0 = degp0.reshape(NP, 1)
    p1 = degp1.reshape(NP, 1)

    y = _k1()(xp, W1, p0, p1)
    parts = _msg_kernel()(y, srcp, dstp, ewp, z2)
    y = _kmid()(parts[0], parts[1], y, p0, p1, b1.reshape(1, D), W2)
    parts = _msg_kernel()(y, srcp, dstp, ewp, z2)
    y = _kmid()(parts[0], parts[1], y, p0, p1, b2.reshape(1, D), W3)
    parts = _msg_kernel()(y, srcp, dstp, ewp, z2)
    out = _kfin()(parts[0], parts[1], y, p0, p1, b3.reshape(1, D))
    return out[:N]


# speedup vs baseline: 8.5644x; 1.3113x over previous
import functools

import jax
import jax.numpy as jnp
from jax import lax
from jax.experimental import pallas as pl
from jax.experimental.pallas import tpu as pltpu
from jax.experimental.pallas import tpu_sc as plsc

N = 10000
NP = 10240
E = 320000
D = 128
C = 64
RPT = 160
HC = RPT // 4
EP = 32 * RPT * C
RB = 2048


def _deg_body(dst_hbm, ew_hbm, z1_hbm, degp0_hbm, degp1_hbm, degsh, didx, ewv):
    c = lax.axis_index("c")
    s = lax.axis_index("s")
    wid = c * 16 + s
    pltpu.sync_copy(z1_hbm.at[pl.ds(s * 640, 640)], degsh.at[pl.ds(s * 640, 640)])
    plsc.subcore_barrier()
    base = wid * RPT
    pltpu.sync_copy(dst_hbm.at[pl.ds(base, RPT)], didx)
    pltpu.sync_copy(ew_hbm.at[pl.ds(base, RPT)], ewv)

    def step(k, carry):
        pltpu.sync_copy(ewv.at[k], degsh.at[didx.at[k]], add=True)
        return carry

    lax.fori_loop(0, RPT, step, 0)
    plsc.subcore_barrier()

    @pl.when(c == 0)
    def _():
        pltpu.sync_copy(degsh.at[pl.ds(s * 640, 640)],
                        degp0_hbm.at[pl.ds(s * 640, 640)])

    @pl.when(c == 1)
    def _():
        pltpu.sync_copy(degsh.at[pl.ds(s * 640, 640)],
                        degp1_hbm.at[pl.ds(s * 640, 640)])


_DNUMS = lax.GatherDimensionNumbers(
    offset_dims=(), collapsed_slice_dims=(0,), start_index_map=(0,))


def _bcast(v, l):
    return lax.gather(v, jnp.full((16, 1), l, jnp.int32), _DNUMS, (1,),
                      mode=lax.GatherScatterMode.PROMISE_IN_BOUNDS)


NA = 10112


def _msg_body(y_hbm, src_hbm, dst_hbm, ew_hbm, z2_hbm, parts_hbm,
              accsh, sidx, didx, ewv, g0, g1, g2, g3,
              gsem0, gsem1, gsem2, gsem3, ssem0, ssem1, ssem2, ssem3):
    c = lax.axis_index("c")
    s = lax.axis_index("s")
    wid = c * 16 + s
    gbufs = (g0, g1, g2, g3)
    gsems = (gsem0, gsem1, gsem2, gsem3)
    ssems = (ssem0, ssem1, ssem2, ssem3)

    pltpu.sync_copy(z2_hbm.at[pl.ds(0, 632)], accsh.at[pl.ds(s * 632, 632)])
    plsc.subcore_barrier()

    def scale_chunk(kl, buf):
        def grp(g, c2):
            ewg = ewv[kl, pl.ds(g * 16, 16)]
            for l in range(16):
                ewj = _bcast(ewg, l)
                j = g * 16 + l
                for kk in range(8):
                    buf[j, pl.ds(kk * 16, 16)] = buf[j, pl.ds(kk * 16, 16)] * ewj
            return c2
        lax.fori_loop(0, C // 16, grp, 0)

    def run_seg(h):
        base = wid * RPT + h * HC
        pltpu.sync_copy(src_hbm.at[pl.ds(base, HC)], sidx)
        pltpu.sync_copy(dst_hbm.at[pl.ds(base, HC)], didx)
        pltpu.sync_copy(ew_hbm.at[pl.ds(base, HC)], ewv)
        pltpu.async_copy(y_hbm.at[sidx.at[0]], gbufs[0], gsems[0])
        pltpu.async_copy(y_hbm.at[sidx.at[1]], gbufs[1], gsems[1])

        def quad(p, carry):
            for q in range(4):
                kl = 4 * p + q
                buf = gbufs[q]
                pltpu.make_async_copy(y_hbm.at[pl.ds(0, C)], buf, gsems[q]).wait()
                scale_chunk(kl, buf)
                pltpu.async_copy(buf, accsh.at[didx.at[kl]], ssems[q], add=True)
                bn = (q + 2) % 4
                if q < 2:
                    @pl.when(p >= 1)
                    def _():
                        pltpu.make_async_copy(
                            gbufs[bn], accsh.at[pl.ds(0, C)], ssems[bn]).wait()
                        pltpu.async_copy(
                            y_hbm.at[sidx.at[kl + 2]], gbufs[bn], gsems[bn])

                    @pl.when(p == 0)
                    def _():
                        pltpu.async_copy(
                            y_hbm.at[sidx.at[kl + 2]], gbufs[bn], gsems[bn])
                else:
                    @pl.when(p <= (HC // 4) - 2)
                    def _():
                        pltpu.make_async_copy(
                            gbufs[bn], accsh.at[pl.ds(0, C)], ssems[bn]).wait()
                        pltpu.async_copy(
                            y_hbm.at[sidx.at[kl + 2]], gbufs[bn], gsems[bn])
            return carry

        lax.fori_loop(0, HC // 4, quad, 0)
        for b in range(4):
            pltpu.make_async_copy(gbufs[b], accsh.at[pl.ds(0, C)], ssems[b]).wait()

    run_seg(0)
    run_seg(1)
    run_seg(2)
    run_seg(3)
    plsc.subcore_barrier()
    pltpu.sync_copy(accsh.at[pl.ds(s * 632, 632)],
                    parts_hbm.at[c, pl.ds(s * 632, 632)])


@functools.cache
def _deg_kernel():
    mesh = plsc.VectorSubcoreMesh(core_axis_name="c", subcore_axis_name="s")
    return pl.kernel(
        _deg_body,
        out_type=(jax.ShapeDtypeStruct((NP,), jnp.float32),
                  jax.ShapeDtypeStruct((NP,), jnp.float32)),
        mesh=mesh,
        scratch_types=[
            pltpu.VMEM_SHARED((NP,), jnp.float32),
            pltpu.VMEM((RPT, C), jnp.int32),
            pltpu.VMEM((RPT, C), jnp.float32),
        ],
    )


@functools.cache
def _msg_kernel():
    mesh = plsc.VectorSubcoreMesh(core_axis_name="c", subcore_axis_name="s")
    return pl.kernel(
        _msg_body,
        out_type=jax.ShapeDtypeStruct((2, NP, D), jnp.float32),
        mesh=mesh,
        scratch_types=[
            pltpu.VMEM_SHARED((NA, D), jnp.float32),
            pltpu.VMEM((HC, C), jnp.int32),
            pltpu.VMEM((HC, C), jnp.int32),
            pltpu.VMEM((HC, C), jnp.float32),
            pltpu.VMEM((C, D), jnp.float32),
            pltpu.VMEM((C, D), jnp.float32),
            pltpu.VMEM((C, D), jnp.float32),
            pltpu.VMEM((C, D), jnp.float32),
        ] + [pltpu.SemaphoreType.DMA] * 8,
    )


def _k1_body(x_ref, w_ref, p0_ref, p1_ref, y_ref):
    dis = lax.rsqrt(p0_ref[...] + p1_ref[...] + 1.0)
    y_ref[...] = jnp.dot(x_ref[...], w_ref[...],
                         preferred_element_type=jnp.float32) * dis


def _kmid_body(a0_ref, a1_ref, yp_ref, p0_ref, p1_ref, b_ref, w_ref, y_ref):
    dis = lax.rsqrt(p0_ref[...] + p1_ref[...] + 1.0)
    h = jnp.maximum(
        dis * (a0_ref[...] + a1_ref[...] + yp_ref[...]) + b_ref[...], 0.0)
    y_ref[...] = jnp.dot(h, w_ref[...],
                         preferred_element_type=jnp.float32) * dis


def _kfin_body(a0_ref, a1_ref, yp_ref, p0_ref, p1_ref, b_ref, o_ref):
    dis = lax.rsqrt(p0_ref[...] + p1_ref[...] + 1.0)
    o_ref[...] = dis * (a0_ref[...] + a1_ref[...] + yp_ref[...]) + b_ref[...]


_row_spec = pl.BlockSpec((RB, D), lambda i: (i, 0))
_col_spec = pl.BlockSpec((RB, 1), lambda i: (i, 0))
_w_spec = pl.BlockSpec((D, D), lambda i: (0, 0))
_b_spec = pl.BlockSpec((1, D), lambda i: (0, 0))
_GRID = (NP // RB,)
_out_nd = jax.ShapeDtypeStruct((NP, D), jnp.float32)


@functools.cache
def _k1():
    return pl.pallas_call(
        _k1_body, grid=_GRID,
        in_specs=[_row_spec, _w_spec, _col_spec, _col_spec],
        out_specs=_row_spec, out_shape=_out_nd)


@functools.cache
def _kmid():
    return pl.pallas_call(
        _kmid_body, grid=_GRID,
        in_specs=[_row_spec, _row_spec, _row_spec, _col_spec, _col_spec,
                  _b_spec, _w_spec],
        out_specs=_row_spec, out_shape=_out_nd)


@functools.cache
def _kfin():
    return pl.pallas_call(
        _kfin_body, grid=_GRID,
        in_specs=[_row_spec, _row_spec, _row_spec, _col_spec, _col_spec,
                  _b_spec],
        out_specs=_row_spec, out_shape=_out_nd)


def kernel(x, edge_index, edge_weight, W1, b1, W2, b2, W3, b3):
    src, dst = edge_index[0], edge_index[1]
    pad_e = EP - E
    srcp = jnp.concatenate([src, jnp.zeros((pad_e,), src.dtype)]).reshape(-1, C)
    dstp = jnp.concatenate([dst, jnp.zeros((pad_e,), dst.dtype)]).reshape(-1, C)
    ewp = jnp.concatenate(
        [edge_weight, jnp.zeros((pad_e,), edge_weight.dtype)]).reshape(-1, C)
    xp = jnp.pad(x, ((0, NP - N), (0, 0)))
    z1 = jnp.zeros((NP,), jnp.float32)
    z2 = jnp.zeros((640, D), jnp.float32)

    degp0, degp1 = _deg_kernel()(dstp, ewp, z1)
    p0 = degp0.reshape(NP, 1)
    p1 = degp1.reshape(NP, 1)

    y = _k1()(xp, W1, p0, p1)
    parts = _msg_kernel()(y, srcp, dstp, ewp, z2)
    y = _kmid()(parts[0], parts[1], y, p0, p1, b1.reshape(1, D), W2)
    parts = _msg_kernel()(y, srcp, dstp, ewp, z2)
    y = _kmid()(parts[0], parts[1], y, p0, p1, b2.reshape(1, D), W3)
    parts = _msg_kernel()(y, srcp, dstp, ewp, z2)
    out = _kfin()(parts[0], parts[1], y, p0, p1, b3.reshape(1, D))
    return out[:N]


# speedup vs baseline: 9.0927x; 1.0617x over previous
import functools

import jax
import jax.numpy as jnp
from jax import lax
from jax.experimental import pallas as pl
from jax.experimental.pallas import tpu as pltpu
from jax.experimental.pallas import tpu_sc as plsc

N = 10000
NP = 10240
E = 320000
D = 128
C = 64
RPT = 160
HC = RPT // 4
EP = 32 * RPT * C
RB = 2048


def _deg_body(dst_hbm, ew_hbm, z1_hbm, degp0_hbm, degp1_hbm, degsh, didx, ewv):
    c = lax.axis_index("c")
    s = lax.axis_index("s")
    wid = c * 16 + s
    pltpu.sync_copy(z1_hbm.at[pl.ds(s * 640, 640)], degsh.at[pl.ds(s * 640, 640)])
    plsc.subcore_barrier()
    base = wid * RPT
    pltpu.sync_copy(dst_hbm.at[pl.ds(base, RPT)], didx)
    pltpu.sync_copy(ew_hbm.at[pl.ds(base, RPT)], ewv)

    def step(k, carry):
        pltpu.sync_copy(ewv.at[k], degsh.at[didx.at[k]], add=True)
        return carry

    lax.fori_loop(0, RPT, step, 0)
    plsc.subcore_barrier()

    @pl.when(c == 0)
    def _():
        pltpu.sync_copy(degsh.at[pl.ds(s * 640, 640)],
                        degp0_hbm.at[pl.ds(s * 640, 640)])

    @pl.when(c == 1)
    def _():
        pltpu.sync_copy(degsh.at[pl.ds(s * 640, 640)],
                        degp1_hbm.at[pl.ds(s * 640, 640)])


_DNUMS = lax.GatherDimensionNumbers(
    offset_dims=(), collapsed_slice_dims=(0,), start_index_map=(0,))


def _bcast(v, l):
    return lax.gather(v, jnp.full((16, 1), l, jnp.int32), _DNUMS, (1,),
                      mode=lax.GatherScatterMode.PROMISE_IN_BOUNDS)


NA = 10112

HC = 32
RPT0 = 224
RPT1 = 96


def _msg_body(y_hbm, src_hbm, dst_hbm, ew_hbm, z2_hbm, parts_hbm,
              accsh, sidx, didx, ewv, g0, g1, g2, g3,
              gsem0, gsem1, gsem2, gsem3, ssem0, ssem1, ssem2, ssem3):
    c = lax.axis_index("c")
    s = lax.axis_index("s")
    gbufs = (g0, g1, g2, g3)
    gsems = (gsem0, gsem1, gsem2, gsem3)
    ssems = (ssem0, ssem1, ssem2, ssem3)

    pltpu.sync_copy(z2_hbm.at[pl.ds(0, 632)], accsh.at[pl.ds(s * 632, 632)])
    plsc.subcore_barrier()

    nseg = jnp.where(c == 0, RPT0 // HC, RPT1 // HC)
    cbase = jnp.where(c == 0, s * RPT0, 16 * RPT0 + s * RPT1)

    def scale_chunk(kl, buf):
        def grp(g, c2):
            ewg = ewv[kl, pl.ds(g * 16, 16)]
            for l in range(16):
                ewj = _bcast(ewg, l)
                j = g * 16 + l
                for kk in range(8):
                    buf[j, pl.ds(kk * 16, 16)] = buf[j, pl.ds(kk * 16, 16)] * ewj
            return c2
        lax.fori_loop(0, C // 16, grp, 0)

    def seg(h, carry):
        base = cbase + h * HC
        pltpu.sync_copy(src_hbm.at[pl.ds(base, HC)], sidx)
        pltpu.sync_copy(dst_hbm.at[pl.ds(base, HC)], didx)
        pltpu.sync_copy(ew_hbm.at[pl.ds(base, HC)], ewv)
        pltpu.async_copy(y_hbm.at[sidx.at[0]], gbufs[0], gsems[0])
        pltpu.async_copy(y_hbm.at[sidx.at[1]], gbufs[1], gsems[1])

        def quad(p, carry2):
            for q in range(4):
                kl = 4 * p + q
                buf = gbufs[q]
                pltpu.make_async_copy(y_hbm.at[pl.ds(0, C)], buf, gsems[q]).wait()
                scale_chunk(kl, buf)
                pltpu.async_copy(buf, accsh.at[didx.at[kl]], ssems[q], add=True)
                bn = (q + 2) % 4
                if q < 2:
                    @pl.when(p >= 1)
                    def _():
                        pltpu.make_async_copy(
                            gbufs[bn], accsh.at[pl.ds(0, C)], ssems[bn]).wait()
                        pltpu.async_copy(
                            y_hbm.at[sidx.at[kl + 2]], gbufs[bn], gsems[bn])

                    @pl.when(p == 0)
                    def _():
                        pltpu.async_copy(
                            y_hbm.at[sidx.at[kl + 2]], gbufs[bn], gsems[bn])
                else:
                    @pl.when(p <= (HC // 4) - 2)
                    def _():
                        pltpu.make_async_copy(
                            gbufs[bn], accsh.at[pl.ds(0, C)], ssems[bn]).wait()
                        pltpu.async_copy(
                            y_hbm.at[sidx.at[kl + 2]], gbufs[bn], gsems[bn])
            return carry2

        lax.fori_loop(0, HC // 4, quad, 0)
        for b in range(4):
            pltpu.make_async_copy(gbufs[b], accsh.at[pl.ds(0, C)], ssems[b]).wait()
        return carry

    lax.fori_loop(0, nseg, seg, 0)
    plsc.subcore_barrier()
    pltpu.sync_copy(accsh.at[pl.ds(s * 632, 632)],
                    parts_hbm.at[c, pl.ds(s * 632, 632)])


@functools.cache
def _deg_kernel():
    mesh = plsc.VectorSubcoreMesh(core_axis_name="c", subcore_axis_name="s")
    return pl.kernel(
        _deg_body,
        out_type=(jax.ShapeDtypeStruct((NP,), jnp.float32),
                  jax.ShapeDtypeStruct((NP,), jnp.float32)),
        mesh=mesh,
        scratch_types=[
            pltpu.VMEM_SHARED((NP,), jnp.float32),
            pltpu.VMEM((RPT, C), jnp.int32),
            pltpu.VMEM((RPT, C), jnp.float32),
        ],
    )


@functools.cache
def _msg_kernel():
    mesh = plsc.VectorSubcoreMesh(core_axis_name="c", subcore_axis_name="s")
    return pl.kernel(
        _msg_body,
        out_type=jax.ShapeDtypeStruct((2, NP, D), jnp.float32),
        mesh=mesh,
        scratch_types=[
            pltpu.VMEM_SHARED((NA, D), jnp.float32),
            pltpu.VMEM((HC, C), jnp.int32),
            pltpu.VMEM((HC, C), jnp.int32),
            pltpu.VMEM((HC, C), jnp.float32),
            pltpu.VMEM((C, D), jnp.float32),
            pltpu.VMEM((C, D), jnp.float32),
            pltpu.VMEM((C, D), jnp.float32),
            pltpu.VMEM((C, D), jnp.float32),
        ] + [pltpu.SemaphoreType.DMA] * 8,
    )


def _k1_body(x_ref, w_ref, p0_ref, p1_ref, y_ref):
    dis = lax.rsqrt(p0_ref[...] + p1_ref[...] + 1.0)
    y_ref[...] = jnp.dot(x_ref[...], w_ref[...],
                         preferred_element_type=jnp.float32) * dis


def _kmid_body(a0_ref, a1_ref, yp_ref, p0_ref, p1_ref, b_ref, w_ref, y_ref):
    dis = lax.rsqrt(p0_ref[...] + p1_ref[...] + 1.0)
    h = jnp.maximum(
        dis * (a0_ref[...] + a1_ref[...] + yp_ref[...]) + b_ref[...], 0.0)
    y_ref[...] = jnp.dot(h, w_ref[...],
                         preferred_element_type=jnp.float32) * dis


def _kfin_body(a0_ref, a1_ref, yp_ref, p0_ref, p1_ref, b_ref, o_ref):
    dis = lax.rsqrt(p0_ref[...] + p1_ref[...] + 1.0)
    o_ref[...] = dis * (a0_ref[...] + a1_ref[...] + yp_ref[...]) + b_ref[...]


_row_spec = pl.BlockSpec((RB, D), lambda i: (i, 0))
_col_spec = pl.BlockSpec((RB, 1), lambda i: (i, 0))
_w_spec = pl.BlockSpec((D, D), lambda i: (0, 0))
_b_spec = pl.BlockSpec((1, D), lambda i: (0, 0))
_GRID = (NP // RB,)
_out_nd = jax.ShapeDtypeStruct((NP, D), jnp.float32)


@functools.cache
def _k1():
    return pl.pallas_call(
        _k1_body, grid=_GRID,
        in_specs=[_row_spec, _w_spec, _col_spec, _col_spec],
        out_specs=_row_spec, out_shape=_out_nd)


@functools.cache
def _kmid():
    return pl.pallas_call(
        _kmid_body, grid=_GRID,
        in_specs=[_row_spec, _row_spec, _row_spec, _col_spec, _col_spec,
                  _b_spec, _w_spec],
        out_specs=_row_spec, out_shape=_out_nd)


@functools.cache
def _kfin():
    return pl.pallas_call(
        _kfin_body, grid=_GRID,
        in_specs=[_row_spec, _row_spec, _row_spec, _col_spec, _col_spec,
                  _b_spec],
        out_specs=_row_spec, out_shape=_out_nd)


def kernel(x, edge_index, edge_weight, W1, b1, W2, b2, W3, b3):
    src, dst = edge_index[0], edge_index[1]
    pad_e = EP - E
    srcp = jnp.concatenate([src, jnp.zeros((pad_e,), src.dtype)]).reshape(-1, C)
    dstp = jnp.concatenate([dst, jnp.zeros((pad_e,), dst.dtype)]).reshape(-1, C)
    ewp = jnp.concatenate(
        [edge_weight, jnp.zeros((pad_e,), edge_weight.dtype)]).reshape(-1, C)
    xp = jnp.pad(x, ((0, NP - N), (0, 0)))
    z1 = jnp.zeros((NP,), jnp.float32)
    z2 = jnp.zeros((640, D), jnp.float32)

    degp0, degp1 = _deg_kernel()(dstp, ewp, z1)
    p0 = degp0.reshape(NP, 1)
    p1 = degp1.reshape(NP, 1)

    y = _k1()(xp, W1, p0, p1)
    parts = _msg_kernel()(y, srcp, dstp, ewp, z2)
    y = _kmid()(parts[0], parts[1], y, p0, p1, b1.reshape(1, D), W2)
    parts = _msg_kernel()(y, srcp, dstp, ewp, z2)
    y = _kmid()(parts[0], parts[1], y, p0, p1, b2.reshape(1, D), W3)
    parts = _msg_kernel()(y, srcp, dstp, ewp, z2)
    out = _kfin()(parts[0], parts[1], y, p0, p1, b3.reshape(1, D))
    return out[:N]


# speedup vs baseline: 21.3907x; 2.3525x over previous
import functools

import jax
import jax.numpy as jnp
from jax import lax
from jax.experimental import pallas as pl
from jax.experimental.pallas import tpu as pltpu
from jax.experimental.pallas import tpu_sc as plsc

N = 10000
NP = 10240
E = 320000
D = 128
C = 64
RPT = 160
HC = RPT // 4
EP = 32 * RPT * C
RB = 2048


def _deg_body(dst_hbm, ew_hbm, z1_hbm, degp0_hbm, degp1_hbm, degsh, didx, ewv,
              dsem):
    c = lax.axis_index("c")
    s = lax.axis_index("s")
    wid = c * 16 + s
    pltpu.sync_copy(z1_hbm.at[pl.ds(s * 640, 640)], degsh.at[pl.ds(s * 640, 640)])
    plsc.subcore_barrier()
    base = wid * RPT
    pltpu.sync_copy(dst_hbm.at[pl.ds(base, RPT)], didx)
    pltpu.sync_copy(ew_hbm.at[pl.ds(base, RPT)], ewv)

    def blk(gg, carry):
        for i in range(16):
            pltpu.async_copy(ewv.at[gg * 16 + i], degsh.at[didx.at[gg * 16 + i]],
                             dsem, add=True)
        for i in range(16):
            pltpu.make_async_copy(ewv.at[0], degsh.at[pl.ds(0, C)], dsem).wait()
        return carry

    lax.fori_loop(0, RPT // 16, blk, 0)
    plsc.subcore_barrier()

    @pl.when(c == 0)
    def _():
        pltpu.sync_copy(degsh.at[pl.ds(s * 640, 640)],
                        degp0_hbm.at[pl.ds(s * 640, 640)])

    @pl.when(c == 1)
    def _():
        pltpu.sync_copy(degsh.at[pl.ds(s * 640, 640)],
                        degp1_hbm.at[pl.ds(s * 640, 640)])


_DNUMS = lax.GatherDimensionNumbers(
    offset_dims=(), collapsed_slice_dims=(0,), start_index_map=(0,))


def _bcast(v, l):
    return lax.gather(v, jnp.full((16, 1), l, jnp.int32), _DNUMS, (1,),
                      mode=lax.GatherScatterMode.PROMISE_IN_BOUNDS)


NA = 10112

HC = 32
RPT0 = 160
RPT1 = 160


def _msg_body(y_hbm, src_hbm, dst_hbm, ew_hbm, z2_hbm, parts_hbm,
              accsh, sidx, didx, ewv, g0, g1, g2, g3,
              gsem0, gsem1, gsem2, gsem3, ssem0, ssem1, ssem2, ssem3):
    c = lax.axis_index("c")
    s = lax.axis_index("s")
    gbufs = (g0, g1, g2, g3)
    gsems = (gsem0, gsem1, gsem2, gsem3)
    ssems = (ssem0, ssem1, ssem2, ssem3)

    pltpu.sync_copy(z2_hbm.at[pl.ds(0, 632)], accsh.at[pl.ds(s * 632, 632)])
    plsc.subcore_barrier()

    nseg = jnp.where(c == 0, RPT0 // HC, RPT1 // HC)
    cbase = jnp.where(c == 0, s * RPT0, 16 * RPT0 + s * RPT1)

    def scale_chunk(kl, buf):
        def grp(g, c2):
            ewg = ewv[kl, pl.ds(g * 16, 16)]
            for l in range(16):
                ewj = _bcast(ewg, l)
                j = g * 16 + l
                for kk in range(8):
                    buf[j, pl.ds(kk * 16, 16)] = buf[j, pl.ds(kk * 16, 16)] * ewj
            return c2
        lax.fori_loop(0, C // 16, grp, 0)

    def seg(h, carry):
        base = cbase + h * HC
        pltpu.sync_copy(src_hbm.at[pl.ds(base, HC)], sidx)
        pltpu.sync_copy(dst_hbm.at[pl.ds(base, HC)], didx)
        pltpu.sync_copy(ew_hbm.at[pl.ds(base, HC)], ewv)
        pltpu.async_copy(y_hbm.at[sidx.at[0]], gbufs[0], gsems[0])
        pltpu.async_copy(y_hbm.at[sidx.at[1]], gbufs[1], gsems[1])

        def quad(p, carry2):
            for q in range(4):
                kl = 4 * p + q
                buf = gbufs[q]
                pltpu.make_async_copy(y_hbm.at[pl.ds(0, C)], buf, gsems[q]).wait()
                scale_chunk(kl, buf)
                pltpu.async_copy(buf, accsh.at[didx.at[kl]], ssems[q], add=True)
                bn = (q + 2) % 4
                if q < 2:
                    @pl.when(p >= 1)
                    def _():
                        pltpu.make_async_copy(
                            gbufs[bn], accsh.at[pl.ds(0, C)], ssems[bn]).wait()
                        pltpu.async_copy(
                            y_hbm.at[sidx.at[kl + 2]], gbufs[bn], gsems[bn])

                    @pl.when(p == 0)
                    def _():
                        pltpu.async_copy(
                            y_hbm.at[sidx.at[kl + 2]], gbufs[bn], gsems[bn])
                else:
                    @pl.when(p <= (HC // 4) - 2)
                    def _():
                        pltpu.make_async_copy(
                            gbufs[bn], accsh.at[pl.ds(0, C)], ssems[bn]).wait()
                        pltpu.async_copy(
                            y_hbm.at[sidx.at[kl + 2]], gbufs[bn], gsems[bn])
            return carry2

        lax.fori_loop(0, HC // 4, quad, 0)
        for b in range(4):
            pltpu.make_async_copy(gbufs[b], accsh.at[pl.ds(0, C)], ssems[b]).wait()
        return carry

    lax.fori_loop(0, nseg, seg, 0)
    plsc.subcore_barrier()
    pltpu.sync_copy(accsh.at[pl.ds(s * 632, 632)],
                    parts_hbm.at[c, pl.ds(s * 632, 632)])


@functools.cache
def _deg_kernel():
    mesh = plsc.VectorSubcoreMesh(core_axis_name="c", subcore_axis_name="s")
    return pl.kernel(
        _deg_body,
        out_type=(jax.ShapeDtypeStruct((NP,), jnp.float32),
                  jax.ShapeDtypeStruct((NP,), jnp.float32)),
        mesh=mesh,
        scratch_types=[
            pltpu.VMEM_SHARED((NP,), jnp.float32),
            pltpu.VMEM((RPT, C), jnp.int32),
            pltpu.VMEM((RPT, C), jnp.float32),
            pltpu.SemaphoreType.DMA,
        ],
    )


@functools.cache
def _msg_kernel():
    mesh = plsc.VectorSubcoreMesh(core_axis_name="c", subcore_axis_name="s")
    return pl.kernel(
        _msg_body,
        out_type=jax.ShapeDtypeStruct((2, NP, D), jnp.float32),
        mesh=mesh,
        scratch_types=[
            pltpu.VMEM_SHARED((NA, D), jnp.float32),
            pltpu.VMEM((HC, C), jnp.int32),
            pltpu.VMEM((HC, C), jnp.int32),
            pltpu.VMEM((HC, C), jnp.float32),
            pltpu.VMEM((C, D), jnp.float32),
            pltpu.VMEM((C, D), jnp.float32),
            pltpu.VMEM((C, D), jnp.float32),
            pltpu.VMEM((C, D), jnp.float32),
        ] + [pltpu.SemaphoreType.DMA] * 8,
    )


def _k1_body(x_ref, w_ref, p0_ref, p1_ref, y_ref):
    dis = lax.rsqrt(p0_ref[...] + p1_ref[...] + 1.0)
    y_ref[...] = jnp.dot(x_ref[...], w_ref[...],
                         preferred_element_type=jnp.float32) * dis


def _kmid_body(a0_ref, a1_ref, yp_ref, p0_ref, p1_ref, b_ref, w_ref, y_ref):
    dis = lax.rsqrt(p0_ref[...] + p1_ref[...] + 1.0)
    h = jnp.maximum(
        dis * (a0_ref[...] + a1_ref[...] + yp_ref[...]) + b_ref[...], 0.0)
    y_ref[...] = jnp.dot(h, w_ref[...],
                         preferred_element_type=jnp.float32) * dis


def _kfin_body(a0_ref, a1_ref, yp_ref, p0_ref, p1_ref, b_ref, o_ref):
    dis = lax.rsqrt(p0_ref[...] + p1_ref[...] + 1.0)
    o_ref[...] = dis * (a0_ref[...] + a1_ref[...] + yp_ref[...]) + b_ref[...]


_row_spec = pl.BlockSpec((RB, D), lambda i: (i, 0))
_col_spec = pl.BlockSpec((RB, 1), lambda i: (i, 0))
_w_spec = pl.BlockSpec((D, D), lambda i: (0, 0))
_b_spec = pl.BlockSpec((1, D), lambda i: (0, 0))
_GRID = (NP // RB,)
_out_nd = jax.ShapeDtypeStruct((NP, D), jnp.float32)


@functools.cache
def _k1():
    return pl.pallas_call(
        _k1_body, grid=_GRID,
        in_specs=[_row_spec, _w_spec, _col_spec, _col_spec],
        out_specs=_row_spec, out_shape=_out_nd)


@functools.cache
def _kmid():
    return pl.pallas_call(
        _kmid_body, grid=_GRID,
        in_specs=[_row_spec, _row_spec, _row_spec, _col_spec, _col_spec,
                  _b_spec, _w_spec],
        out_specs=_row_spec, out_shape=_out_nd)


@functools.cache
def _kfin():
    return pl.pallas_call(
        _kfin_body, grid=_GRID,
        in_specs=[_row_spec, _row_spec, _row_spec, _col_spec, _col_spec,
                  _b_spec],
        out_specs=_row_spec, out_shape=_out_nd)


def kernel(x, edge_index, edge_weight, W1, b1, W2, b2, W3, b3):
    src, dst = edge_index[0], edge_index[1]
    pad_e = EP - E
    spread = jnp.arange(pad_e, dtype=src.dtype) % N
    srcp = jnp.concatenate([src, spread]).reshape(-1, C)
    dstp = jnp.concatenate([dst, spread]).reshape(-1, C)
    ewp = jnp.concatenate(
        [edge_weight, jnp.zeros((pad_e,), edge_weight.dtype)]).reshape(-1, C)
    xp = jnp.pad(x, ((0, NP - N), (0, 0)))
    z1 = jnp.zeros((NP,), jnp.float32)
    z2 = jnp.zeros((640, D), jnp.float32)

    degp0, degp1 = _deg_kernel()(dstp, ewp, z1)
    p0 = degp0.reshape(NP, 1)
    p1 = degp1.reshape(NP, 1)

    y = _k1()(xp, W1, p0, p1)
    parts = _msg_kernel()(y, srcp, dstp, ewp, z2)
    y = _kmid()(parts[0], parts[1], y, p0, p1, b1.reshape(1, D), W2)
    parts = _msg_kernel()(y, srcp, dstp, ewp, z2)
    y = _kmid()(parts[0], parts[1], y, p0, p1, b2.reshape(1, D), W3)
    parts = _msg_kernel()(y, srcp, dstp, ewp, z2)
    out = _kfin()(parts[0], parts[1], y, p0, p1, b3.reshape(1, D))
    return out[:N]
